# baseline (speedup 1.0000x reference)
import functools

import jax
import jax.numpy as jnp
from jax import lax
from jax.experimental import pallas as pl
from jax.experimental.pallas import tpu as pltpu
from jax.experimental.pallas import tpu_sc as plsc

N, E, D, C = 10000, 320000, 128, 40
NP = 10240
NC, NS, L = 2, 16, 16
NW = NC * NS
EPW = E // NW
K = 96
NB = 105
EPWP = NB * K
EP = NW * EPWP
DPAD = NP - 8
RPT = NP // NS

@functools.cache
def _sc_degree_kernel():
    mesh = plsc.VectorSubcoreMesh(core_axis_name="c", subcore_axis_name="s",
                                  num_cores=NC, num_subcores=NS)
    return pl.kernel(
        _sc_degree_body,
        out_type=jax.ShapeDtypeStruct((NC, NS, NP), jnp.float32),
        mesh=mesh,
        compiler_params=pltpu.CompilerParams(needs_layout_passes=False),
        scratch_types=[
            pltpu.VMEM((EPW,), jnp.int32),
            pltpu.VMEM((NP,), jnp.float32),
        ],
    )


def _sc_degree(dst):
    return _sc_degree_kernel()(dst)


def _sc_degree_body(dst_hbm, out_hbm, idx_v, deg_v):
    cid = lax.axis_index("c")
    sid = lax.axis_index("s")
    wid = sid * NC + cid
    pltpu.sync_copy(dst_hbm.at[pl.ds(wid * EPW, EPW)], idx_v)

    def _zero(j, carry):
        deg_v[pl.ds(j * L, L)] = jnp.zeros((L,), jnp.float32)
        return carry
    lax.fori_loop(0, NP // L, _zero, 0)

    one = jnp.ones((L,), jnp.float32)
    def _hist(j, carry):
        idx16 = idx_v[pl.ds(j * L, L)]
        plsc.addupdate_scatter(deg_v, [idx16], one)
        return carry
    lax.fori_loop(0, EPW // L, _hist, 0)
    pltpu.sync_copy(deg_v, out_hbm.at[cid, sid])


@functools.cache
def _sc_agg_kernel():
    mesh = plsc.VectorSubcoreMesh(core_axis_name="c", subcore_axis_name="s",
                                  num_cores=NC, num_subcores=NS)
    return pl.kernel(
        _sc_agg_body,
        out_type=jax.ShapeDtypeStruct((NC, NP, D), jnp.float32),
        mesh=mesh,
        scratch_types=[
            pltpu.VMEM((EPWP,), jnp.int32),
            pltpu.VMEM((NB, K), jnp.int32),
            pltpu.VMEM((K, D), jnp.float32),
            pltpu.VMEM((K, D), jnp.float32),
            pltpu.VMEM_SHARED((NP, D), jnp.float32),
            pltpu.SemaphoreType.DMA,
            pltpu.SemaphoreType.DMA,
            pltpu.SemaphoreType.DMA,
            pltpu.SemaphoreType.DMA,
        ],
    )


def _sc_agg(p, src, dst2, zeros):
    return _sc_agg_kernel()(p, src, dst2, zeros)


def _sc_agg_body(p_hbm, src_hbm, dst2_hbm, zeros_hbm, out_hbm,
                 isrc_v, idst_v, rows0, rows1, acc_sh, gs0, gs1, ss0, ss1):
    cid = lax.axis_index("c")
    sid = lax.axis_index("s")
    wid = sid * NC + cid
    pltpu.sync_copy(zeros_hbm.at[pl.ds(sid * RPT, RPT)],
                    acc_sh.at[pl.ds(sid * RPT, RPT)])
    pltpu.sync_copy(src_hbm.at[pl.ds(wid * EPWP, EPWP)], isrc_v)
    pltpu.sync_copy(dst2_hbm.at[wid], idst_v)
    plsc.subcore_barrier()

    def _start(j, rows, sem):
        pltpu.async_copy(p_hbm.at[isrc_v.at[pl.ds(j * K, K)]], rows, sem)

    def _wait_g(rows, sem):
        pltpu.make_async_copy(p_hbm.at[pl.ds(0, K)], rows, sem).wait()

    def _sstart(j, rows, sem):
        pltpu.async_copy(rows, acc_sh.at[idst_v.at[j]], sem, add=True)

    def _wait_s(j, rows, sem):
        pltpu.make_async_copy(rows, acc_sh.at[idst_v.at[j]], sem).wait()

    _start(0, rows0, gs0)
    def _body(i, carry):
        j0 = 2 * i

        @pl.when(i > 0)
        def _drain():
            _wait_s(j0 - 1, rows1, ss1)

        _start(j0 + 1, rows1, gs1)
        _wait_g(rows0, gs0)
        _sstart(j0, rows0, ss0)
        _wait_s(j0, rows0, ss0)
        _start(j0 + 2, rows0, gs0)
        _wait_g(rows1, gs1)
        _sstart(j0 + 1, rows1, ss1)
        return carry
    lax.fori_loop(0, NB // 2, _body, 0)
    _wait_s(NB - 2, rows1, ss1)
    _wait_g(rows0, gs0)
    _sstart(NB - 1, rows0, ss0)
    _wait_s(NB - 1, rows0, ss0)

    plsc.subcore_barrier()
    pltpu.sync_copy(acc_sh.at[pl.ds(sid * RPT, RPT)],
                    out_hbm.at[cid, pl.ds(sid * RPT, RPT)])


RB1 = 2048
RB = 2000


def _t1_body(degp_ref, x_ref, dinv_ref, p1_ref):
    d = lax.dot_general(degp_ref[...], jnp.ones((NW, 1), jnp.float32),
                        (((0,), (0,)), ((), ())),
                        preferred_element_type=jnp.float32) + 1.0
    dinv = lax.rsqrt(jnp.maximum(d, 1.0))
    dinv_ref[...] = dinv
    p1_ref[...] = x_ref[...] * dinv


def _t1_call(degp, x):
    return pl.pallas_call(
        _t1_body,
        grid=(NP // RB1,),
        in_specs=[
            pl.BlockSpec((NW, RB1), lambda i: (0, i)),
            pl.BlockSpec((RB1, D), lambda i: (i, 0)),
        ],
        out_specs=[
            pl.BlockSpec((RB1, 1), lambda i: (i, 0)),
            pl.BlockSpec((RB1, D), lambda i: (i, 0)),
        ],
        out_shape=[
            jax.ShapeDtypeStruct((NP, 1), jnp.float32),
            jax.ShapeDtypeStruct((NP, D), jnp.float32),
        ],
    )(degp, x)


def _t2_body(acc_ref, p1_ref, dinv_ref, W1_ref, b1_ref, p2_ref):
    dinv = dinv_ref[...]
    t = (acc_ref[0] + acc_ref[1] + p1_ref[...]) * dinv
    h1 = jnp.maximum(jnp.dot(t, W1_ref[...],
                             preferred_element_type=jnp.float32)
                     + b1_ref[...], 0.0)
    p2_ref[...] = h1 * dinv


def _t2_call(acc1, p1, dinv, W1, b1):
    return pl.pallas_call(
        _t2_body,
        grid=(N // RB,),
        in_specs=[
            pl.BlockSpec((2, RB, D), lambda i: (0, i, 0)),
            pl.BlockSpec((RB, D), lambda i: (i, 0)),
            pl.BlockSpec((RB, 1), lambda i: (i, 0)),
            pl.BlockSpec((D, D), lambda i: (0, 0)),
            pl.BlockSpec((1, D), lambda i: (0, 0)),
        ],
        out_specs=pl.BlockSpec((RB, D), lambda i: (i, 0)),
        out_shape=jax.ShapeDtypeStruct((N, D), jnp.float32),
    )(acc1, p1, dinv, W1, b1.reshape(1, D))


def _t3_body(acc_ref, p2_ref, dinv_ref, W2_ref, b2_ref, Wfc_ref, bfc_ref,
             h_ref, y_ref):
    dinv = dinv_ref[...]
    t = (acc_ref[0] + acc_ref[1] + p2_ref[...]) * dinv
    h = jnp.dot(t, W2_ref[...], preferred_element_type=jnp.float32) + b2_ref[...]
    h_ref[...] = h
    y_ref[...] = (jnp.dot(jnp.maximum(h, 0.0), Wfc_ref[...],
                          preferred_element_type=jnp.float32)
                  + bfc_ref[...])


def _t3_call(acc2, p2, dinv, W2, b2, Wfc, bfc):
    return pl.pallas_call(
        _t3_body,
        grid=(N // RB,),
        in_specs=[
            pl.BlockSpec((2, RB, D), lambda i: (0, i, 0)),
            pl.BlockSpec((RB, D), lambda i: (i, 0)),
            pl.BlockSpec((RB, 1), lambda i: (i, 0)),
            pl.BlockSpec((D, D), lambda i: (0, 0)),
            pl.BlockSpec((1, D), lambda i: (0, 0)),
            pl.BlockSpec((D, C), lambda i: (0, 0)),
            pl.BlockSpec((1, C), lambda i: (0, 0)),
        ],
        out_specs=[
            pl.BlockSpec((RB, D), lambda i: (i, 0)),
            pl.BlockSpec((RB, C), lambda i: (i, 0)),
        ],
        out_shape=[
            jax.ShapeDtypeStruct((N, D), jnp.float32),
            jax.ShapeDtypeStruct((N, C), jnp.float32),
        ],
    )(acc2, p2, dinv, W2, b2.reshape(1, D), Wfc, bfc.reshape(1, C))


def kernel(x, edge_index, W1, b1, W2, b2, Wfc, bfc):
    src = edge_index[0]
    dst = edge_index[1]
    srcp = jnp.pad(src, (0, EP - E))
    dst2 = jnp.pad(dst, (0, EP - E), constant_values=DPAD)
    dst2 = dst2.reshape(NW, NB, K)
    zD = jnp.zeros((NP, D), jnp.float32)

    degp = _sc_degree(dst)
    degp = degp.reshape(NW, NP)
    xp = jnp.pad(x, ((0, NP - N), (0, 0)))
    dinv, p1 = _t1_call(degp, xp)
    acc1 = _sc_agg(p1, srcp, dst2, zD)
    p2 = _t2_call(acc1, p1, dinv, W1, b1)
    acc2 = _sc_agg(p2, srcp, dst2, zD)
    h, y = _t3_call(acc2, p2, dinv, W2, b2, Wfc, bfc)
    return (h, y)


# speedup vs baseline: 1.0003x; 1.0003x over previous
import functools

import jax
import jax.numpy as jnp
from jax import lax
from jax.experimental import pallas as pl
from jax.experimental.pallas import tpu as pltpu
from jax.experimental.pallas import tpu_sc as plsc

N, E, D, C = 10000, 320000, 128, 40
NP = 10240
NC, NS, L = 2, 16, 16
NW = NC * NS
EPW = E // NW
K = 96
NB = 105
EPWP = NB * K
EP = NW * EPWP
DPAD = NP - 8
RPT = NP // NS

@functools.cache
def _sc_degree_kernel():
    mesh = plsc.VectorSubcoreMesh(core_axis_name="c", subcore_axis_name="s",
                                  num_cores=NC, num_subcores=NS)
    return pl.kernel(
        _sc_degree_body,
        out_type=jax.ShapeDtypeStruct((NC, NS, NP), jnp.float32),
        mesh=mesh,
        compiler_params=pltpu.CompilerParams(needs_layout_passes=False),
        scratch_types=[
            pltpu.VMEM((EPW,), jnp.int32),
            pltpu.VMEM((NP,), jnp.float32),
        ],
    )


def _sc_degree(dst):
    return _sc_degree_kernel()(dst)


def _sc_degree_body(dst_hbm, out_hbm, idx_v, deg_v):
    cid = lax.axis_index("c")
    sid = lax.axis_index("s")
    wid = sid * NC + cid
    pltpu.sync_copy(dst_hbm.at[pl.ds(wid * EPW, EPW)], idx_v)

    def _zero(j, carry):
        deg_v[pl.ds(j * L, L)] = jnp.zeros((L,), jnp.float32)
        return carry
    lax.fori_loop(0, NP // L, _zero, 0)

    one = jnp.ones((L,), jnp.float32)
    def _hist(j, carry):
        idx16 = idx_v[pl.ds(j * L, L)]
        plsc.addupdate_scatter(deg_v, [idx16], one)
        return carry
    lax.fori_loop(0, EPW // L, _hist, 0)
    pltpu.sync_copy(deg_v, out_hbm.at[cid, sid])


@functools.cache
def _sc_agg_kernel():
    mesh = plsc.VectorSubcoreMesh(core_axis_name="c", subcore_axis_name="s",
                                  num_cores=NC, num_subcores=NS)
    return pl.kernel(
        _sc_agg_body,
        out_type=jax.ShapeDtypeStruct((NC, NP, D), jnp.float32),
        mesh=mesh,
        scratch_types=[
            pltpu.VMEM((EPWP,), jnp.int32),
            pltpu.VMEM((NB, K), jnp.int32),
            pltpu.VMEM((K, D), jnp.float32),
            pltpu.VMEM((K, D), jnp.float32),
            pltpu.VMEM_SHARED((NP, D), jnp.float32),
            pltpu.SemaphoreType.DMA,
            pltpu.SemaphoreType.DMA,
            pltpu.SemaphoreType.DMA,
            pltpu.SemaphoreType.DMA,
        ],
    )


def _sc_agg(p, src, dst2, zeros):
    return _sc_agg_kernel()(p, src, dst2, zeros)


def _sc_agg_body(p_hbm, src_hbm, dst2_hbm, zeros_hbm, out_hbm,
                 isrc_v, idst_v, rows0, rows1, acc_sh, gs0, gs1, ss0, ss1):
    cid = lax.axis_index("c")
    sid = lax.axis_index("s")
    wid = sid * NC + cid
    pltpu.sync_copy(zeros_hbm.at[pl.ds(sid * RPT, RPT)],
                    acc_sh.at[pl.ds(sid * RPT, RPT)])
    pltpu.sync_copy(src_hbm.at[pl.ds(wid * EPWP, EPWP)], isrc_v)
    pltpu.sync_copy(dst2_hbm.at[wid], idst_v)
    plsc.subcore_barrier()

    def _start(j, rows, sem):
        pltpu.async_copy(p_hbm.at[isrc_v.at[pl.ds(j * K, K)]], rows, sem)

    def _wait_g(rows, sem):
        pltpu.make_async_copy(p_hbm.at[pl.ds(0, K)], rows, sem).wait()

    def _sstart(j, rows, sem):
        pltpu.async_copy(rows, acc_sh.at[idst_v.at[j]], sem, add=True)

    def _wait_s(j, rows, sem):
        pltpu.make_async_copy(rows, acc_sh.at[idst_v.at[j]], sem).wait()

    _start(0, rows0, gs0)
    def _body(i, carry):
        j0 = 2 * i
        _start(j0 + 1, rows1, gs1)
        _wait_g(rows0, gs0)
        _sstart(j0, rows0, ss0)
        _wait_s(j0, rows0, ss0)
        _start(j0 + 2, rows0, gs0)
        _wait_g(rows1, gs1)
        _sstart(j0 + 1, rows1, ss1)
        _wait_s(j0 + 1, rows1, ss1)
        return carry
    lax.fori_loop(0, NB // 2, _body, 0)
    _wait_g(rows0, gs0)
    _sstart(NB - 1, rows0, ss0)
    _wait_s(NB - 1, rows0, ss0)

    plsc.subcore_barrier()
    pltpu.sync_copy(acc_sh.at[pl.ds(sid * RPT, RPT)],
                    out_hbm.at[cid, pl.ds(sid * RPT, RPT)])


RB1 = 2048
RB = 2000


def _t1_body(degp_ref, x_ref, dinv_ref, p1_ref):
    d = lax.dot_general(degp_ref[...], jnp.ones((NW, 1), jnp.float32),
                        (((0,), (0,)), ((), ())),
                        preferred_element_type=jnp.float32) + 1.0
    dinv = lax.rsqrt(jnp.maximum(d, 1.0))
    dinv_ref[...] = dinv
    p1_ref[...] = x_ref[...] * dinv


def _t1_call(degp, x):
    return pl.pallas_call(
        _t1_body,
        grid=(NP // RB1,),
        in_specs=[
            pl.BlockSpec((NW, RB1), lambda i: (0, i)),
            pl.BlockSpec((RB1, D), lambda i: (i, 0)),
        ],
        out_specs=[
            pl.BlockSpec((RB1, 1), lambda i: (i, 0)),
            pl.BlockSpec((RB1, D), lambda i: (i, 0)),
        ],
        out_shape=[
            jax.ShapeDtypeStruct((NP, 1), jnp.float32),
            jax.ShapeDtypeStruct((NP, D), jnp.float32),
        ],
    )(degp, x)


def _t2_body(acc_ref, p1_ref, dinv_ref, W1_ref, b1_ref, p2_ref):
    dinv = dinv_ref[...]
    t = (acc_ref[0] + acc_ref[1] + p1_ref[...]) * dinv
    h1 = jnp.maximum(jnp.dot(t, W1_ref[...],
                             preferred_element_type=jnp.float32)
                     + b1_ref[...], 0.0)
    p2_ref[...] = h1 * dinv


def _t2_call(acc1, p1, dinv, W1, b1):
    return pl.pallas_call(
        _t2_body,
        grid=(N // RB,),
        in_specs=[
            pl.BlockSpec((2, RB, D), lambda i: (0, i, 0)),
            pl.BlockSpec((RB, D), lambda i: (i, 0)),
            pl.BlockSpec((RB, 1), lambda i: (i, 0)),
            pl.BlockSpec((D, D), lambda i: (0, 0)),
            pl.BlockSpec((1, D), lambda i: (0, 0)),
        ],
        out_specs=pl.BlockSpec((RB, D), lambda i: (i, 0)),
        out_shape=jax.ShapeDtypeStruct((N, D), jnp.float32),
    )(acc1, p1, dinv, W1, b1.reshape(1, D))


def _t3_body(acc_ref, p2_ref, dinv_ref, W2_ref, b2_ref, Wfc_ref, bfc_ref,
             h_ref, y_ref):
    dinv = dinv_ref[...]
    t = (acc_ref[0] + acc_ref[1] + p2_ref[...]) * dinv
    h = jnp.dot(t, W2_ref[...], preferred_element_type=jnp.float32) + b2_ref[...]
    h_ref[...] = h
    y_ref[...] = (jnp.dot(jnp.maximum(h, 0.0), Wfc_ref[...],
                          preferred_element_type=jnp.float32)
                  + bfc_ref[...])


def _t3_call(acc2, p2, dinv, W2, b2, Wfc, bfc):
    return pl.pallas_call(
        _t3_body,
        grid=(N // RB,),
        in_specs=[
            pl.BlockSpec((2, RB, D), lambda i: (0, i, 0)),
            pl.BlockSpec((RB, D), lambda i: (i, 0)),
            pl.BlockSpec((RB, 1), lambda i: (i, 0)),
            pl.BlockSpec((D, D), lambda i: (0, 0)),
            pl.BlockSpec((1, D), lambda i: (0, 0)),
            pl.BlockSpec((D, C), lambda i: (0, 0)),
            pl.BlockSpec((1, C), lambda i: (0, 0)),
        ],
        out_specs=[
            pl.BlockSpec((RB, D), lambda i: (i, 0)),
            pl.BlockSpec((RB, C), lambda i: (i, 0)),
        ],
        out_shape=[
            jax.ShapeDtypeStruct((N, D), jnp.float32),
            jax.ShapeDtypeStruct((N, C), jnp.float32),
        ],
    )(acc2, p2, dinv, W2, b2.reshape(1, D), Wfc, bfc.reshape(1, C))


def kernel(x, edge_index, W1, b1, W2, b2, Wfc, bfc):
    src = edge_index[0]
    dst = edge_index[1]
    srcp = jnp.pad(src, (0, EP - E))
    dst2 = jnp.pad(dst, (0, EP - E), constant_values=DPAD)
    dst2 = dst2.reshape(NW, NB, K)
    zD = jnp.zeros((NP, D), jnp.float32)

    degp = _sc_degree(dst)
    degp = degp.reshape(NW, NP)
    xp = jnp.pad(x, ((0, NP - N), (0, 0)))
    dinv, p1 = _t1_call(degp, xp)
    acc1 = _sc_agg(p1, srcp, dst2, zD)
    p2 = _t2_call(acc1, p1, dinv, W1, b1)
    acc2 = _sc_agg(p2, srcp, dst2, zD)
    h, y = _t3_call(acc2, p2, dinv, W2, b2, Wfc, bfc)
    return (h, y)


# speedup vs baseline: 1.7628x; 1.7622x over previous
import functools

import jax
import jax.numpy as jnp
from jax import lax
from jax.experimental import pallas as pl
from jax.experimental.pallas import tpu as pltpu
from jax.experimental.pallas import tpu_sc as plsc

N, E, D, C = 10000, 320000, 128, 40
NP = 10240
NC, NS, L = 2, 16, 16
NW = NC * NS
EPW = E // NW
K = 80
NB = EPW // K
RPT = NP // NS

@functools.cache
def _sc_degree_kernel():
    mesh = plsc.VectorSubcoreMesh(core_axis_name="c", subcore_axis_name="s",
                                  num_cores=NC, num_subcores=NS)
    return pl.kernel(
        _sc_degree_body,
        out_type=jax.ShapeDtypeStruct((NC, NS, NP), jnp.float32),
        mesh=mesh,
        compiler_params=pltpu.CompilerParams(needs_layout_passes=False),
        scratch_types=[
            pltpu.VMEM((EPW,), jnp.int32),
            pltpu.VMEM((NP,), jnp.float32),
        ],
    )


def _sc_degree(dst):
    return _sc_degree_kernel()(dst)


def _sc_degree_body(dst_hbm, out_hbm, idx_v, deg_v):
    cid = lax.axis_index("c")
    sid = lax.axis_index("s")
    wid = sid * NC + cid
    pltpu.sync_copy(dst_hbm.at[pl.ds(wid * EPW, EPW)], idx_v)

    def _zero(j, carry):
        deg_v[pl.ds(j * L, L)] = jnp.zeros((L,), jnp.float32)
        return carry
    lax.fori_loop(0, NP // L, _zero, 0)

    one = jnp.ones((L,), jnp.float32)
    def _hist(j, carry):
        idx16 = idx_v[pl.ds(j * L, L)]
        plsc.addupdate_scatter(deg_v, [idx16], one)
        return carry
    lax.fori_loop(0, EPW // L, _hist, 0)
    pltpu.sync_copy(deg_v, out_hbm.at[cid, sid])


@functools.cache
def _sc_agg_kernel():
    mesh = plsc.VectorSubcoreMesh(core_axis_name="c", subcore_axis_name="s",
                                  num_cores=NC, num_subcores=NS)
    return pl.kernel(
        _sc_agg_body,
        out_type=jax.ShapeDtypeStruct((NC, NP, D), jnp.float32),
        mesh=mesh,
        scratch_types=[
            pltpu.VMEM((EPW,), jnp.int32),
            pltpu.VMEM((NB, K), jnp.int32),
            pltpu.VMEM((K, D), jnp.float32),
            pltpu.VMEM((K, D), jnp.float32),
            pltpu.VMEM_SHARED((NP, D), jnp.float32),
            pltpu.SemaphoreType.DMA,
            pltpu.SemaphoreType.DMA,
            pltpu.SemaphoreType.DMA,
            pltpu.SemaphoreType.DMA,
        ],
    )


def _sc_agg(p, src, dst2, zeros):
    return _sc_agg_kernel()(p, src, dst2, zeros)


def _sc_agg_body(p_hbm, src_hbm, dst2_hbm, zeros_hbm, out_hbm,
                 isrc_v, idst_v, rows0, rows1, acc_sh, gs0, gs1, ss0, ss1):
    cid = lax.axis_index("c")
    sid = lax.axis_index("s")
    wid = sid * NC + cid
    pltpu.sync_copy(zeros_hbm.at[pl.ds(sid * RPT, RPT)],
                    acc_sh.at[pl.ds(sid * RPT, RPT)])
    pltpu.sync_copy(src_hbm.at[pl.ds(wid * EPW, EPW)], isrc_v)
    pltpu.sync_copy(dst2_hbm.at[wid], idst_v)
    plsc.subcore_barrier()

    def _start(j, rows, sem):
        pltpu.async_copy(p_hbm.at[isrc_v.at[pl.ds(j * K, K)]], rows, sem)

    def _wait_g(rows, sem):
        pltpu.make_async_copy(p_hbm.at[pl.ds(0, K)], rows, sem).wait()

    def _sstart(j, rows, sem):
        pltpu.async_copy(rows, acc_sh.at[idst_v.at[j]], sem, add=True)

    def _wait_s(j, rows, sem):
        pltpu.make_async_copy(rows, acc_sh.at[idst_v.at[j]], sem).wait()

    _start(0, rows0, gs0)
    def _body(i, carry):
        j0 = 2 * i
        _start(j0 + 1, rows1, gs1)
        _wait_g(rows0, gs0)
        _sstart(j0, rows0, ss0)
        _wait_s(j0, rows0, ss0)
        _start(j0 + 2, rows0, gs0)
        _wait_g(rows1, gs1)
        _sstart(j0 + 1, rows1, ss1)
        _wait_s(j0 + 1, rows1, ss1)
        return carry
    lax.fori_loop(0, NB // 2, _body, 0)
    _wait_g(rows0, gs0)
    _sstart(NB - 1, rows0, ss0)
    _wait_s(NB - 1, rows0, ss0)

    plsc.subcore_barrier()
    pltpu.sync_copy(acc_sh.at[pl.ds(sid * RPT, RPT)],
                    out_hbm.at[cid, pl.ds(sid * RPT, RPT)])


RB1 = 2048
RB = 2000


def _t1_body(degp_ref, x_ref, dinv_ref, p1_ref):
    d = lax.dot_general(degp_ref[...], jnp.ones((NW, 1), jnp.float32),
                        (((0,), (0,)), ((), ())),
                        preferred_element_type=jnp.float32) + 1.0
    dinv = lax.rsqrt(jnp.maximum(d, 1.0))
    dinv_ref[...] = dinv
    p1_ref[...] = x_ref[...] * dinv


def _t1_call(degp, x):
    return pl.pallas_call(
        _t1_body,
        grid=(NP // RB1,),
        in_specs=[
            pl.BlockSpec((NW, RB1), lambda i: (0, i)),
            pl.BlockSpec((RB1, D), lambda i: (i, 0)),
        ],
        out_specs=[
            pl.BlockSpec((RB1, 1), lambda i: (i, 0)),
            pl.BlockSpec((RB1, D), lambda i: (i, 0)),
        ],
        out_shape=[
            jax.ShapeDtypeStruct((NP, 1), jnp.float32),
            jax.ShapeDtypeStruct((NP, D), jnp.float32),
        ],
    )(degp, x)


def _t2_body(acc_ref, p1_ref, dinv_ref, W1_ref, b1_ref, p2_ref):
    dinv = dinv_ref[...]
    t = (acc_ref[0] + acc_ref[1] + p1_ref[...]) * dinv
    h1 = jnp.maximum(jnp.dot(t, W1_ref[...],
                             preferred_element_type=jnp.float32)
                     + b1_ref[...], 0.0)
    p2_ref[...] = h1 * dinv


def _t2_call(acc1, p1, dinv, W1, b1):
    return pl.pallas_call(
        _t2_body,
        grid=(N // RB,),
        in_specs=[
            pl.BlockSpec((2, RB, D), lambda i: (0, i, 0)),
            pl.BlockSpec((RB, D), lambda i: (i, 0)),
            pl.BlockSpec((RB, 1), lambda i: (i, 0)),
            pl.BlockSpec((D, D), lambda i: (0, 0)),
            pl.BlockSpec((1, D), lambda i: (0, 0)),
        ],
        out_specs=pl.BlockSpec((RB, D), lambda i: (i, 0)),
        out_shape=jax.ShapeDtypeStruct((N, D), jnp.float32),
    )(acc1, p1, dinv, W1, b1.reshape(1, D))


def _t3_body(acc_ref, p2_ref, dinv_ref, W2_ref, b2_ref, Wfc_ref, bfc_ref,
             h_ref, y_ref):
    dinv = dinv_ref[...]
    t = (acc_ref[0] + acc_ref[1] + p2_ref[...]) * dinv
    h = jnp.dot(t, W2_ref[...], preferred_element_type=jnp.float32) + b2_ref[...]
    h_ref[...] = h
    y_ref[...] = (jnp.dot(jnp.maximum(h, 0.0), Wfc_ref[...],
                          preferred_element_type=jnp.float32)
                  + bfc_ref[...])


def _t3_call(acc2, p2, dinv, W2, b2, Wfc, bfc):
    return pl.pallas_call(
        _t3_body,
        grid=(N // RB,),
        in_specs=[
            pl.BlockSpec((2, RB, D), lambda i: (0, i, 0)),
            pl.BlockSpec((RB, D), lambda i: (i, 0)),
            pl.BlockSpec((RB, 1), lambda i: (i, 0)),
            pl.BlockSpec((D, D), lambda i: (0, 0)),
            pl.BlockSpec((1, D), lambda i: (0, 0)),
            pl.BlockSpec((D, C), lambda i: (0, 0)),
            pl.BlockSpec((1, C), lambda i: (0, 0)),
        ],
        out_specs=[
            pl.BlockSpec((RB, D), lambda i: (i, 0)),
            pl.BlockSpec((RB, C), lambda i: (i, 0)),
        ],
        out_shape=[
            jax.ShapeDtypeStruct((N, D), jnp.float32),
            jax.ShapeDtypeStruct((N, C), jnp.float32),
        ],
    )(acc2, p2, dinv, W2, b2.reshape(1, D), Wfc, bfc.reshape(1, C))


def kernel(x, edge_index, W1, b1, W2, b2, Wfc, bfc):
    src = edge_index[0]
    dst = edge_index[1]
    dst2 = dst.reshape(NW, NB, K)
    zD = jnp.zeros((NP, D), jnp.float32)

    degp = _sc_degree(dst)
    degp = degp.reshape(NW, NP)
    xp = jnp.pad(x, ((0, NP - N), (0, 0)))
    dinv, p1 = _t1_call(degp, xp)
    acc1 = _sc_agg(p1, src, dst2, zD)
    p2 = _t2_call(acc1, p1, dinv, W1, b1)
    acc2 = _sc_agg(p2, src, dst2, zD)
    h, y = _t3_call(acc2, p2, dinv, W2, b2, Wfc, bfc)
    return (h, y)
